# baseline (speedup 1.0000x reference)
import functools

import jax
import jax.numpy as jnp
from jax import lax
from jax.experimental import pallas as pl
from jax.experimental.pallas import tpu as pltpu
from jax.experimental.pallas import tpu_sc as plsc

E = 8
TOPK = 2
NHOPS = 2
CAP = 256
D = 1024
DFF = 2048
V = 16384
T = 2048

NC = 2
NS = 16
NW = NC * NS


def _sc_gather_body(idx_hbm, table_hbm, out_hbm, idx_v, rows_v, sem, *, b_per_w):
    wid = lax.axis_index("s") * NC + lax.axis_index("c")
    base = wid * b_per_w
    pltpu.sync_copy(idx_hbm.at[pl.ds(base, b_per_w)], idx_v)
    pltpu.async_copy(table_hbm.at[idx_v], rows_v, sem).wait()
    pltpu.sync_copy(rows_v, out_hbm.at[pl.ds(base, b_per_w)])


def _sc_gather_rows(table, idx, b):
    n, d = table.shape
    b_per_w = b // NW
    mesh = plsc.VectorSubcoreMesh(core_axis_name="c", subcore_axis_name="s")
    k = pl.kernel(
        functools.partial(_sc_gather_body, b_per_w=b_per_w),
        out_type=jax.ShapeDtypeStruct((b, d), jnp.float32),
        mesh=mesh,
        scratch_types=[
            pltpu.VMEM((b_per_w,), jnp.int32),
            pltpu.VMEM((b_per_w, d), jnp.float32),
            pltpu.SemaphoreType.DMA,
        ],
    )
    return k(idx, table)


def _sc_gather2_body(idx0_hbm, idx1_hbm, table_hbm, out0_hbm, out1_hbm,
                     idx0_v, idx1_v, rows0_v, rows1_v, sem0, sem1, *, b_per_w):
    wid = lax.axis_index("s") * NC + lax.axis_index("c")
    base = wid * b_per_w
    half = b_per_w // 2
    pltpu.sync_copy(idx0_hbm.at[pl.ds(base, b_per_w)], idx0_v)
    pltpu.sync_copy(idx1_hbm.at[pl.ds(base, b_per_w)], idx1_v)
    for ch in range(2):
        a0 = pltpu.async_copy(table_hbm.at[idx0_v.at[pl.ds(ch * half, half)]],
                              rows0_v, sem0)
        a1 = pltpu.async_copy(table_hbm.at[idx1_v.at[pl.ds(ch * half, half)]],
                              rows1_v, sem1)
        a0.wait()
        pltpu.sync_copy(rows0_v, out0_hbm.at[pl.ds(base + ch * half, half)])
        a1.wait()
        pltpu.sync_copy(rows1_v, out1_hbm.at[pl.ds(base + ch * half, half)])


def _sc_gather_rows2(table, idx0, idx1, b):
    n, d = table.shape
    b_per_w = b // NW
    mesh = plsc.VectorSubcoreMesh(core_axis_name="c", subcore_axis_name="s")
    k = pl.kernel(
        functools.partial(_sc_gather2_body, b_per_w=b_per_w),
        out_type=(jax.ShapeDtypeStruct((b, d), jnp.float32),
                  jax.ShapeDtypeStruct((b, d), jnp.float32)),
        mesh=mesh,
        scratch_types=[
            pltpu.VMEM((b_per_w,), jnp.int32),
            pltpu.VMEM((b_per_w,), jnp.int32),
            pltpu.VMEM((b_per_w // 2, d), jnp.float32),
            pltpu.VMEM((b_per_w // 2, d), jnp.float32),
            pltpu.SemaphoreType.DMA,
            pltpu.SemaphoreType.DMA,
        ],
    )
    return k(idx0, idx1, table)


_ROUTE_OUT_SPECS = [
    pl.BlockSpec((E, T), lambda: (0, 0)),
    pl.BlockSpec((1, T), lambda: (0, 0)),
    pl.BlockSpec((1, T), lambda: (0, 0)),
    pl.BlockSpec((1, T), lambda: (0, 0)),
    pl.BlockSpec((1, T), lambda: (0, 0)),
    pl.BlockSpec((1, T), lambda: (0, 0)),
]
_ROUTE_OUT_SHAPES = [
    jax.ShapeDtypeStruct((E, T), jnp.int32),
    jax.ShapeDtypeStruct((1, T), jnp.int32),
    jax.ShapeDtypeStruct((1, T), jnp.int32),
    jax.ShapeDtypeStruct((1, T), jnp.float32),
    jax.ShapeDtypeStruct((1, T), jnp.float32),
    jax.ShapeDtypeStruct((1, T), jnp.float32),
]


def _mlp_body(x_ref, w1_ref, w3_ref, w2_ref, out_ref):
    x = x_ref[...]
    t1 = jnp.dot(x, w1_ref[0], preferred_element_type=jnp.float32)
    t3 = jnp.dot(x, w3_ref[0], preferred_element_type=jnp.float32)
    a = t1 * jax.nn.sigmoid(t1) * t3
    contrib = jnp.dot(a, w2_ref[0], preferred_element_type=jnp.float32)

    @pl.when(pl.program_id(1) == 0)
    def _init():
        out_ref[...] = contrib

    @pl.when(pl.program_id(1) != 0)
    def _acc():
        out_ref[...] += contrib


def _tc_moe_mlp(xin, W1, W3, W2):
    dff_blk = 2048
    kk = DFF // dff_blk
    return pl.pallas_call(
        _mlp_body,
        grid=(E, kk),
        in_specs=[
            pl.BlockSpec((CAP, D), lambda e, k: (e, 0)),
            pl.BlockSpec((1, D, dff_blk), lambda e, k: (e, 0, k)),
            pl.BlockSpec((1, D, dff_blk), lambda e, k: (e, 0, k)),
            pl.BlockSpec((1, dff_blk, D), lambda e, k: (e, k, 0)),
        ],
        out_specs=pl.BlockSpec((CAP, D), lambda e, k: (e, 0)),
        out_shape=jax.ShapeDtypeStruct((E * CAP, D), jnp.float32),
        compiler_params=pltpu.CompilerParams(
            dimension_semantics=("parallel", "arbitrary")),
    )(xin, W1, W3, W2)


def _finalize_head_body(h_ref, g0_ref, g1_ref, sc_ref, w0_ref, w1_ref,
                        g_ref, emb_ref, out_ref, hn_scr):
    @pl.when(pl.program_id(0) == 0)
    def _compute_hn():
        hb = (sc_ref[...] * h_ref[...] + w0_ref[...] * g0_ref[...]
              + w1_ref[...] * g1_ref[...])
        ms = jnp.mean(hb * hb, axis=1, keepdims=True)
        hn_scr[...] = (hb * lax.rsqrt(ms + 1e-6)
                       * g_ref[...]).astype(jnp.bfloat16)

    out_ref[...] = lax.dot_general(
        hn_scr[...], emb_ref[...].astype(jnp.bfloat16),
        (((1,), (1,)), ((), ())),
        preferred_element_type=jnp.float32)


def _tc_finalize_head(h, g0, g1, scale, w0, w1, gvec, embed):
    vb = 512
    return pl.pallas_call(
        _finalize_head_body,
        grid=(V // vb,),
        in_specs=[
            pl.BlockSpec((T, D), lambda j: (0, 0)),
            pl.BlockSpec((T, D), lambda j: (0, 0)),
            pl.BlockSpec((T, D), lambda j: (0, 0)),
            pl.BlockSpec((T, 1), lambda j: (0, 0)),
            pl.BlockSpec((T, 1), lambda j: (0, 0)),
            pl.BlockSpec((T, 1), lambda j: (0, 0)),
            pl.BlockSpec((1, D), lambda j: (0, 0)),
            pl.BlockSpec((vb, D), lambda j: (j, 0)),
        ],
        out_specs=pl.BlockSpec((T, vb), lambda j: (0, j)),
        out_shape=jax.ShapeDtypeStruct((T, V), jnp.float32),
        scratch_shapes=[pltpu.VMEM((T, D), jnp.bfloat16)],
        compiler_params=pltpu.CompilerParams(
            dimension_semantics=("arbitrary",)),
    )(h, g0, g1, scale, w0, w1, gvec.reshape(1, D), embed)


def _lane_prefix_sum(c, iota_t):
    k = 1
    while k < T:
        r = pltpu.roll(c, k, 1)
        c = c + jnp.where(iota_t >= k, r, 0)
        k *= 2
    return c


def _route_write(l, fscat_ref, s0_ref, s1_ref, w0_ref, w1_ref, sc_ref):
    iota_e = lax.broadcasted_iota(jnp.int32, (E, T), 0)
    iota_t = lax.broadcasted_iota(jnp.int32, (E, T), 1)
    m1 = jnp.max(l, axis=0, keepdims=True)
    e0 = jnp.min(jnp.where(l == m1, iota_e, E), axis=0, keepdims=True)
    is0 = iota_e == e0
    l2 = jnp.where(is0, -jnp.inf, l)
    m2 = jnp.max(l2, axis=0, keepdims=True)
    e1 = jnp.min(jnp.where(l2 == m2, iota_e, E), axis=0, keepdims=True)
    is1 = iota_e == e1
    mask = is0 | is1
    p = jnp.exp(l - m1)
    p = p / jnp.sum(p, axis=0, keepdims=True)
    bi = lax.bitcast_convert_type(l, jnp.int32)
    bu = lax.bitcast_convert_type(l, jnp.uint32)
    u = jnp.where(bi >= 0, bu + jnp.uint32(0x80000000), ~bu)
    u = jnp.where(mask, u, jnp.uint32(0))

    def _bit_step(i, acc):
        bit = lax.shift_left(jnp.uint32(1), (31 - i).astype(jnp.uint32))
        cand = acc | bit
        cnt = jnp.sum((u >= cand).astype(jnp.int32), axis=1, keepdims=True)
        return jnp.where(cnt >= CAP, cand, acc)

    acc = lax.fori_loop(0, 32, _bit_step, jnp.zeros((E, 1), jnp.uint32))
    n_above = jnp.sum((u > acc).astype(jnp.int32), axis=1, keepdims=True)
    tie = mask & (u == acc)
    trank = _lane_prefix_sum(tie.astype(jnp.int32), iota_t) - 1
    kept = mask & ((u > acc) | (tie & (trank < CAP - n_above)))
    slot = _lane_prefix_sum(kept.astype(jnp.int32), iota_t) - 1
    local_e = jnp.where(iota_e >= E // 2, iota_e - E // 2, iota_e)
    fscat_ref[...] = jnp.where(kept, local_e * CAP + slot, CAP * E // 2 + iota_t)
    slot_g = iota_e * CAP + slot
    tokid = lax.broadcasted_iota(jnp.int32, (1, T), 1)
    sel0 = is0 & kept
    k0 = jnp.sum(sel0.astype(jnp.int32), axis=0, keepdims=True)
    g0 = jnp.sum(jnp.where(sel0, slot_g, 0), axis=0, keepdims=True)
    w0 = jnp.sum(jnp.where(sel0, p, 0.0), axis=0, keepdims=True)
    sel1 = is1 & kept
    k1 = jnp.sum(sel1.astype(jnp.int32), axis=0, keepdims=True)
    g1 = jnp.sum(jnp.where(sel1, slot_g, 0), axis=0, keepdims=True)
    w1 = jnp.sum(jnp.where(sel1, p, 0.0), axis=0, keepdims=True)
    s0_ref[...] = jnp.where(k0 > 0, g0, tokid)
    s1_ref[...] = jnp.where(k1 > 0, g1, tokid)
    w0_ref[...] = w0
    w1_ref[...] = w1
    sc_ref[...] = 1.0 - w0 - w1


def _logits_route_body(h_ref, wr_ref, fscat_ref, s0_ref, s1_ref,
                       w0_ref, w1_ref, sc_ref):
    l = lax.dot_general(wr_ref[...], h_ref[...], (((0,), (1,)), ((), ())),
                        preferred_element_type=jnp.float32)
    _route_write(l, fscat_ref, s0_ref, s1_ref, w0_ref, w1_ref, sc_ref)


def _tc_logits_route(h, wr):
    return pl.pallas_call(
        _logits_route_body,
        in_specs=[
            pl.BlockSpec((T, D), lambda: (0, 0)),
            pl.BlockSpec((D, E), lambda: (0, 0)),
        ],
        out_specs=_ROUTE_OUT_SPECS,
        out_shape=_ROUTE_OUT_SHAPES,
    )(h, wr)


def _update_route_body(h_ref, g0_ref, g1_ref, si_ref, wi0_ref, wi1_ref,
                       wr_ref, h1_ref, fscat_ref, s0_ref, s1_ref,
                       w0_ref, w1_ref, sc_ref):
    hb = (si_ref[...] * h_ref[...] + wi0_ref[...] * g0_ref[...]
          + wi1_ref[...] * g1_ref[...])
    h1_ref[...] = hb
    l = lax.dot_general(wr_ref[...], hb, (((0,), (1,)), ((), ())),
                        preferred_element_type=jnp.float32)
    _route_write(l, fscat_ref, s0_ref, s1_ref, w0_ref, w1_ref, sc_ref)


def _tc_update_route(h, g0, g1, scale, w0, w1, wr):
    return pl.pallas_call(
        _update_route_body,
        in_specs=[
            pl.BlockSpec((T, D), lambda: (0, 0)),
            pl.BlockSpec((T, D), lambda: (0, 0)),
            pl.BlockSpec((T, D), lambda: (0, 0)),
            pl.BlockSpec((T, 1), lambda: (0, 0)),
            pl.BlockSpec((T, 1), lambda: (0, 0)),
            pl.BlockSpec((T, 1), lambda: (0, 0)),
            pl.BlockSpec((D, E), lambda: (0, 0)),
        ],
        out_specs=[pl.BlockSpec((T, D), lambda: (0, 0))] + _ROUTE_OUT_SPECS,
        out_shape=[jax.ShapeDtypeStruct((T, D), jnp.float32)]
        + _ROUTE_OUT_SHAPES,
    )(h, g0, g1, scale, w0, w1, wr)


def _sc_dispatch_body(fscat_hbm, tokvals_hbm, inits_hbm, h_hbm, out_hbm,
                      idx_v, vals_v, init_v, gidx_v, rows_v, disp_sh, sem):
    c = lax.axis_index("c")
    s = lax.axis_index("s")
    half = CAP * E // 2
    pltpu.sync_copy(inits_hbm.at[pl.ds(c * half + s * 64, 64)], init_v)
    pltpu.sync_copy(init_v, disp_sh.at[pl.ds(s * 64, 64)])
    row = (c * NS + s) * 4
    pltpu.sync_copy(fscat_hbm.at[pl.ds(row, 4)], idx_v)
    pltpu.sync_copy(tokvals_hbm.at[pl.ds(row, 4)], vals_v)
    plsc.subcore_barrier()
    for j in range(4):
        pltpu.sync_copy(vals_v.at[j], disp_sh.at[idx_v.at[j]])
    plsc.subcore_barrier()
    pltpu.sync_copy(disp_sh.at[pl.ds(s * 64, 64)], gidx_v)
    pltpu.async_copy(h_hbm.at[gidx_v], rows_v, sem).wait()
    pltpu.sync_copy(rows_v, out_hbm.at[pl.ds(c * half + s * 64, 64)])


def _sc_dispatch(h, fscat, tokvals, inits):
    mesh = plsc.VectorSubcoreMesh(core_axis_name="c", subcore_axis_name="s")
    k = pl.kernel(
        _sc_dispatch_body,
        out_type=jax.ShapeDtypeStruct((E * CAP, D), jnp.float32),
        mesh=mesh,
        scratch_types=[
            pltpu.VMEM((4, 128), jnp.int32),
            pltpu.VMEM((4, 128), jnp.int32),
            pltpu.VMEM((64,), jnp.int32),
            pltpu.VMEM((64,), jnp.int32),
            pltpu.VMEM((64, D), jnp.float32),
            pltpu.VMEM_SHARED((CAP * E // 2 + T,), jnp.int32),
            pltpu.SemaphoreType.DMA,
        ],
    )
    return k(fscat.reshape(128, 128), tokvals, inits, h)


def kernel(ids, embed, Wr, W1, W3, W2, g):
    ids = ids.astype(jnp.int32)
    iota_r = lax.broadcasted_iota(jnp.int32, (128, 128), 0)
    iota_c = lax.broadcasted_iota(jnp.int32, (128, 128), 1)
    tokvals = (iota_r % (T // 128)) * 128 + iota_c
    inits = jnp.arange(T, dtype=jnp.int32)
    h = _sc_gather_rows(embed, ids, T)
    route = _tc_logits_route(h, Wr[0])
    for hop in range(NHOPS):
        fscat, s0, s1, w0, w1, scale = route
        xin = _sc_dispatch(h, fscat, tokvals, inits)
        eo = _tc_moe_mlp(xin, W1, W3, W2)
        g0, g1 = _sc_gather_rows2(eo, s0.reshape(T), s1.reshape(T), T)
        w0c, w1c, sc = (w0.reshape(T, 1), w1.reshape(T, 1),
                        scale.reshape(T, 1))
        if hop + 1 < NHOPS:
            h, *route = _tc_update_route(h, g0, g1, sc, w0c, w1c, Wr[hop + 1])
        else:
            out = _tc_finalize_head(h, g0, g1, sc, w0c, w1c, g, embed)
    return out


# speedup vs baseline: 1.0160x; 1.0160x over previous
import functools

import jax
import jax.numpy as jnp
from jax import lax
from jax.experimental import pallas as pl
from jax.experimental.pallas import tpu as pltpu
from jax.experimental.pallas import tpu_sc as plsc

E = 8
TOPK = 2
NHOPS = 2
CAP = 256
D = 1024
DFF = 2048
V = 16384
T = 2048

NC = 2
NS = 16
NW = NC * NS


def _sc_gather_body(idx_hbm, table_hbm, out_hbm, idx_v, rows_v, sem, *, b_per_w):
    wid = lax.axis_index("s") * NC + lax.axis_index("c")
    base = wid * b_per_w
    pltpu.sync_copy(idx_hbm.at[pl.ds(base, b_per_w)], idx_v)
    pltpu.async_copy(table_hbm.at[idx_v], rows_v, sem).wait()
    pltpu.sync_copy(rows_v, out_hbm.at[pl.ds(base, b_per_w)])


def _sc_gather_rows(table, idx, b):
    n, d = table.shape
    b_per_w = b // NW
    mesh = plsc.VectorSubcoreMesh(core_axis_name="c", subcore_axis_name="s")
    k = pl.kernel(
        functools.partial(_sc_gather_body, b_per_w=b_per_w),
        out_type=jax.ShapeDtypeStruct((b, d), jnp.float32),
        mesh=mesh,
        scratch_types=[
            pltpu.VMEM((b_per_w,), jnp.int32),
            pltpu.VMEM((b_per_w, d), jnp.float32),
            pltpu.SemaphoreType.DMA,
        ],
    )
    return k(idx, table)


def _sc_gather2_body(idx0_hbm, idx1_hbm, table_hbm, out0_hbm, out1_hbm,
                     idx0_v, idx1_v, rows0_v, rows1_v, sem0, sem1, *, b_per_w):
    wid = lax.axis_index("s") * NC + lax.axis_index("c")
    base = wid * b_per_w
    half = b_per_w // 2
    pltpu.sync_copy(idx0_hbm.at[pl.ds(base, b_per_w)], idx0_v)
    pltpu.sync_copy(idx1_hbm.at[pl.ds(base, b_per_w)], idx1_v)
    for ch in range(2):
        a0 = pltpu.async_copy(table_hbm.at[idx0_v.at[pl.ds(ch * half, half)]],
                              rows0_v, sem0)
        a1 = pltpu.async_copy(table_hbm.at[idx1_v.at[pl.ds(ch * half, half)]],
                              rows1_v, sem1)
        a0.wait()
        pltpu.sync_copy(rows0_v, out0_hbm.at[pl.ds(base + ch * half, half)])
        a1.wait()
        pltpu.sync_copy(rows1_v, out1_hbm.at[pl.ds(base + ch * half, half)])


def _sc_gather_rows2(table, idx0, idx1, b):
    n, d = table.shape
    b_per_w = b // NW
    mesh = plsc.VectorSubcoreMesh(core_axis_name="c", subcore_axis_name="s")
    k = pl.kernel(
        functools.partial(_sc_gather2_body, b_per_w=b_per_w),
        out_type=(jax.ShapeDtypeStruct((b, d), jnp.float32),
                  jax.ShapeDtypeStruct((b, d), jnp.float32)),
        mesh=mesh,
        scratch_types=[
            pltpu.VMEM((b_per_w,), jnp.int32),
            pltpu.VMEM((b_per_w,), jnp.int32),
            pltpu.VMEM((b_per_w // 2, d), jnp.float32),
            pltpu.VMEM((b_per_w // 2, d), jnp.float32),
            pltpu.SemaphoreType.DMA,
            pltpu.SemaphoreType.DMA,
        ],
    )
    return k(idx0, idx1, table)


_ROUTE_OUT_SPECS = [
    pl.BlockSpec((E, T), lambda: (0, 0)),
    pl.BlockSpec((1, T), lambda: (0, 0)),
    pl.BlockSpec((1, T), lambda: (0, 0)),
    pl.BlockSpec((1, T), lambda: (0, 0)),
    pl.BlockSpec((1, T), lambda: (0, 0)),
    pl.BlockSpec((1, T), lambda: (0, 0)),
]
_ROUTE_OUT_SHAPES = [
    jax.ShapeDtypeStruct((E, T), jnp.int32),
    jax.ShapeDtypeStruct((1, T), jnp.int32),
    jax.ShapeDtypeStruct((1, T), jnp.int32),
    jax.ShapeDtypeStruct((1, T), jnp.float32),
    jax.ShapeDtypeStruct((1, T), jnp.float32),
    jax.ShapeDtypeStruct((1, T), jnp.float32),
]


def _mlp_body(x_ref, w1_ref, w3_ref, w2_ref, out_ref):
    x = x_ref[...]
    t1 = jnp.dot(x, w1_ref[0], preferred_element_type=jnp.float32)
    t3 = jnp.dot(x, w3_ref[0], preferred_element_type=jnp.float32)
    a = t1 * jax.nn.sigmoid(t1) * t3
    contrib = jnp.dot(a, w2_ref[0], preferred_element_type=jnp.float32)

    @pl.when(pl.program_id(1) == 0)
    def _init():
        out_ref[...] = contrib

    @pl.when(pl.program_id(1) != 0)
    def _acc():
        out_ref[...] += contrib


def _tc_moe_mlp(xin, W1, W3, W2):
    dff_blk = 1024
    kk = DFF // dff_blk
    return pl.pallas_call(
        _mlp_body,
        grid=(E, kk),
        in_specs=[
            pl.BlockSpec((CAP, D), lambda e, k: (e, 0)),
            pl.BlockSpec((1, D, dff_blk), lambda e, k: (e, 0, k)),
            pl.BlockSpec((1, D, dff_blk), lambda e, k: (e, 0, k)),
            pl.BlockSpec((1, dff_blk, D), lambda e, k: (e, k, 0)),
        ],
        out_specs=pl.BlockSpec((CAP, D), lambda e, k: (e, 0)),
        out_shape=jax.ShapeDtypeStruct((E * CAP, D), jnp.float32),
        compiler_params=pltpu.CompilerParams(
            dimension_semantics=("parallel", "arbitrary")),
    )(xin, W1, W3, W2)


def _finalize_head_body(h_ref, g0_ref, g1_ref, sc_ref, w0_ref, w1_ref,
                        g_ref, emb_ref, out_ref, hn_scr):
    @pl.when(pl.program_id(0) == 0)
    def _compute_hn():
        hb = (sc_ref[...] * h_ref[...] + w0_ref[...] * g0_ref[...]
              + w1_ref[...] * g1_ref[...])
        ms = jnp.mean(hb * hb, axis=1, keepdims=True)
        hn_scr[...] = (hb * lax.rsqrt(ms + 1e-6)
                       * g_ref[...]).astype(jnp.bfloat16)

    out_ref[...] = lax.dot_general(
        hn_scr[...], emb_ref[...].astype(jnp.bfloat16),
        (((1,), (1,)), ((), ())),
        preferred_element_type=jnp.float32)


def _tc_finalize_head(h, g0, g1, scale, w0, w1, gvec, embed):
    vb = 512
    return pl.pallas_call(
        _finalize_head_body,
        grid=(V // vb,),
        in_specs=[
            pl.BlockSpec((T, D), lambda j: (0, 0)),
            pl.BlockSpec((T, D), lambda j: (0, 0)),
            pl.BlockSpec((T, D), lambda j: (0, 0)),
            pl.BlockSpec((T, 1), lambda j: (0, 0)),
            pl.BlockSpec((T, 1), lambda j: (0, 0)),
            pl.BlockSpec((T, 1), lambda j: (0, 0)),
            pl.BlockSpec((1, D), lambda j: (0, 0)),
            pl.BlockSpec((vb, D), lambda j: (j, 0)),
        ],
        out_specs=pl.BlockSpec((T, vb), lambda j: (0, j)),
        out_shape=jax.ShapeDtypeStruct((T, V), jnp.float32),
        scratch_shapes=[pltpu.VMEM((T, D), jnp.bfloat16)],
        compiler_params=pltpu.CompilerParams(
            dimension_semantics=("arbitrary",)),
    )(h, g0, g1, scale, w0, w1, gvec.reshape(1, D), embed)


def _lane_prefix_sum(c, iota_t):
    k = 1
    while k < T:
        r = pltpu.roll(c, k, 1)
        c = c + jnp.where(iota_t >= k, r, 0)
        k *= 2
    return c


def _route_write(l, fscat_ref, s0_ref, s1_ref, w0_ref, w1_ref, sc_ref):
    iota_e = lax.broadcasted_iota(jnp.int32, (E, T), 0)
    iota_t = lax.broadcasted_iota(jnp.int32, (E, T), 1)
    m1 = jnp.max(l, axis=0, keepdims=True)
    e0 = jnp.min(jnp.where(l == m1, iota_e, E), axis=0, keepdims=True)
    is0 = iota_e == e0
    l2 = jnp.where(is0, -jnp.inf, l)
    m2 = jnp.max(l2, axis=0, keepdims=True)
    e1 = jnp.min(jnp.where(l2 == m2, iota_e, E), axis=0, keepdims=True)
    is1 = iota_e == e1
    mask = is0 | is1
    p = jnp.exp(l - m1)
    p = p / jnp.sum(p, axis=0, keepdims=True)
    bi = lax.bitcast_convert_type(l, jnp.int32)
    bu = lax.bitcast_convert_type(l, jnp.uint32)
    u = jnp.where(bi >= 0, bu + jnp.uint32(0x80000000), ~bu)
    u = jnp.where(mask, u, jnp.uint32(0))

    def _bit_step(i, acc):
        bit = lax.shift_left(jnp.uint32(1), (31 - i).astype(jnp.uint32))
        cand = acc | bit
        cnt = jnp.sum((u >= cand).astype(jnp.int32), axis=1, keepdims=True)
        return jnp.where(cnt >= CAP, cand, acc)

    acc = lax.fori_loop(0, 32, _bit_step, jnp.zeros((E, 1), jnp.uint32))
    n_above = jnp.sum((u > acc).astype(jnp.int32), axis=1, keepdims=True)
    tie = mask & (u == acc)
    trank = _lane_prefix_sum(tie.astype(jnp.int32), iota_t) - 1
    kept = mask & ((u > acc) | (tie & (trank < CAP - n_above)))
    slot = _lane_prefix_sum(kept.astype(jnp.int32), iota_t) - 1
    local_e = jnp.where(iota_e >= E // 2, iota_e - E // 2, iota_e)
    fscat_ref[...] = jnp.where(kept, local_e * CAP + slot, CAP * E // 2 + iota_t)
    slot_g = iota_e * CAP + slot
    tokid = lax.broadcasted_iota(jnp.int32, (1, T), 1)
    sel0 = is0 & kept
    k0 = jnp.sum(sel0.astype(jnp.int32), axis=0, keepdims=True)
    g0 = jnp.sum(jnp.where(sel0, slot_g, 0), axis=0, keepdims=True)
    w0 = jnp.sum(jnp.where(sel0, p, 0.0), axis=0, keepdims=True)
    sel1 = is1 & kept
    k1 = jnp.sum(sel1.astype(jnp.int32), axis=0, keepdims=True)
    g1 = jnp.sum(jnp.where(sel1, slot_g, 0), axis=0, keepdims=True)
    w1 = jnp.sum(jnp.where(sel1, p, 0.0), axis=0, keepdims=True)
    s0_ref[...] = jnp.where(k0 > 0, g0, tokid)
    s1_ref[...] = jnp.where(k1 > 0, g1, tokid)
    w0_ref[...] = w0
    w1_ref[...] = w1
    sc_ref[...] = 1.0 - w0 - w1


def _logits_route_body(h_ref, wr_ref, fscat_ref, s0_ref, s1_ref,
                       w0_ref, w1_ref, sc_ref):
    l = lax.dot_general(wr_ref[...], h_ref[...], (((0,), (1,)), ((), ())),
                        preferred_element_type=jnp.float32)
    _route_write(l, fscat_ref, s0_ref, s1_ref, w0_ref, w1_ref, sc_ref)


def _tc_logits_route(h, wr):
    return pl.pallas_call(
        _logits_route_body,
        in_specs=[
            pl.BlockSpec((T, D), lambda: (0, 0)),
            pl.BlockSpec((D, E), lambda: (0, 0)),
        ],
        out_specs=_ROUTE_OUT_SPECS,
        out_shape=_ROUTE_OUT_SHAPES,
    )(h, wr)


def _update_route_body(h_ref, g0_ref, g1_ref, si_ref, wi0_ref, wi1_ref,
                       wr_ref, h1_ref, fscat_ref, s0_ref, s1_ref,
                       w0_ref, w1_ref, sc_ref):
    hb = (si_ref[...] * h_ref[...] + wi0_ref[...] * g0_ref[...]
          + wi1_ref[...] * g1_ref[...])
    h1_ref[...] = hb
    l = lax.dot_general(wr_ref[...], hb, (((0,), (1,)), ((), ())),
                        preferred_element_type=jnp.float32)
    _route_write(l, fscat_ref, s0_ref, s1_ref, w0_ref, w1_ref, sc_ref)


def _tc_update_route(h, g0, g1, scale, w0, w1, wr):
    return pl.pallas_call(
        _update_route_body,
        in_specs=[
            pl.BlockSpec((T, D), lambda: (0, 0)),
            pl.BlockSpec((T, D), lambda: (0, 0)),
            pl.BlockSpec((T, D), lambda: (0, 0)),
            pl.BlockSpec((T, 1), lambda: (0, 0)),
            pl.BlockSpec((T, 1), lambda: (0, 0)),
            pl.BlockSpec((T, 1), lambda: (0, 0)),
            pl.BlockSpec((D, E), lambda: (0, 0)),
        ],
        out_specs=[pl.BlockSpec((T, D), lambda: (0, 0))] + _ROUTE_OUT_SPECS,
        out_shape=[jax.ShapeDtypeStruct((T, D), jnp.float32)]
        + _ROUTE_OUT_SHAPES,
    )(h, g0, g1, scale, w0, w1, wr)


def _sc_dispatch_body(fscat_hbm, tokvals_hbm, inits_hbm, h_hbm, out_hbm,
                      idx_v, vals_v, init_v, gidx_v, rows_v, disp_sh, sem):
    c = lax.axis_index("c")
    s = lax.axis_index("s")
    half = CAP * E // 2
    pltpu.sync_copy(inits_hbm.at[pl.ds(c * half + s * 64, 64)], init_v)
    pltpu.sync_copy(init_v, disp_sh.at[pl.ds(s * 64, 64)])
    row = (c * NS + s) * 4
    pltpu.sync_copy(fscat_hbm.at[pl.ds(row, 4)], idx_v)
    pltpu.sync_copy(tokvals_hbm.at[pl.ds(row, 4)], vals_v)
    plsc.subcore_barrier()
    for j in range(4):
        pltpu.sync_copy(vals_v.at[j], disp_sh.at[idx_v.at[j]])
    plsc.subcore_barrier()
    pltpu.sync_copy(disp_sh.at[pl.ds(s * 64, 64)], gidx_v)
    pltpu.async_copy(h_hbm.at[gidx_v], rows_v, sem).wait()
    pltpu.sync_copy(rows_v, out_hbm.at[pl.ds(c * half + s * 64, 64)])


def _sc_dispatch(h, fscat, tokvals, inits):
    mesh = plsc.VectorSubcoreMesh(core_axis_name="c", subcore_axis_name="s")
    k = pl.kernel(
        _sc_dispatch_body,
        out_type=jax.ShapeDtypeStruct((E * CAP, D), jnp.float32),
        mesh=mesh,
        scratch_types=[
            pltpu.VMEM((4, 128), jnp.int32),
            pltpu.VMEM((4, 128), jnp.int32),
            pltpu.VMEM((64,), jnp.int32),
            pltpu.VMEM((64,), jnp.int32),
            pltpu.VMEM((64, D), jnp.float32),
            pltpu.VMEM_SHARED((CAP * E // 2 + T,), jnp.int32),
            pltpu.SemaphoreType.DMA,
        ],
    )
    return k(fscat.reshape(128, 128), tokvals, inits, h)


def kernel(ids, embed, Wr, W1, W3, W2, g):
    ids = ids.astype(jnp.int32)
    iota_r = lax.broadcasted_iota(jnp.int32, (128, 128), 0)
    iota_c = lax.broadcasted_iota(jnp.int32, (128, 128), 1)
    tokvals = (iota_r % (T // 128)) * 128 + iota_c
    inits = jnp.arange(T, dtype=jnp.int32)
    h = _sc_gather_rows(embed, ids, T)
    route = _tc_logits_route(h, Wr[0])
    for hop in range(NHOPS):
        fscat, s0, s1, w0, w1, scale = route
        xin = _sc_dispatch(h, fscat, tokvals, inits)
        eo = _tc_moe_mlp(xin, W1, W3, W2)
        g0, g1 = _sc_gather_rows2(eo, s0.reshape(T), s1.reshape(T), T)
        w0c, w1c, sc = (w0.reshape(T, 1), w1.reshape(T, 1),
                        scale.reshape(T, 1))
        if hop + 1 < NHOPS:
            h, *route = _tc_update_route(h, g0, g1, sc, w0c, w1c, Wr[hop + 1])
        else:
            out = _tc_finalize_head(h, g0, g1, sc, w0c, w1c, g, embed)
    return out
